# baseline (speedup 1.0000x reference)
import functools

import jax
import jax.numpy as jnp
from jax import lax
from jax.experimental import pallas as pl
from jax.experimental.pallas import tpu as pltpu
from jax.experimental.pallas import tpu_sc as plsc

_N = 10000
_E = 320000
_D = 128

_NC = 2
_NS = 16
_CHUNK = 80
_EDGES_PER_TILE = _E // (_NC * _NS)
_STEPS = _EDGES_PER_TILE // _CHUNK
_N_PAD = 10240
_ROWS_PER_TILE = _N_PAD // _NS
_BLK_CH = 40
_NFULL = _STEPS // _BLK_CH
_TAIL = _STEPS - _NFULL * _BLK_CH


def _sc_scatter_body(hp_hbm, src_hbm, dst_hbm, zeros_hbm, out_hbm,
                     sidx, didx, rows0, rows1, acc, sem0, sem1, isem):
    c = lax.axis_index("c")
    s = lax.axis_index("s")
    w = c * _NS + s
    row_base = s * _ROWS_PER_TILE
    ci = pltpu.async_copy(src_hbm.at[w, pl.ds(0, _BLK_CH)], sidx, isem)
    cj = pltpu.async_copy(dst_hbm.at[w, pl.ds(0, _BLK_CH)], didx, isem)
    pltpu.sync_copy(zeros_hbm.at[pl.ds(row_base, _ROWS_PER_TILE)],
                    acc.at[pl.ds(row_base, _ROWS_PER_TILE)])
    ci.wait()
    cj.wait()
    plsc.subcore_barrier()

    def process(nch):
        pltpu.async_copy(hp_hbm.at[sidx.at[0]], rows0, sem0)

        def pair(i, carry):
            k0 = 2 * i
            pltpu.async_copy(hp_hbm.at[sidx.at[k0 + 1]], rows1, sem1)
            pltpu.make_async_copy(hp_hbm.at[sidx.at[k0]], rows0, sem0).wait()
            pltpu.sync_copy(rows0, acc.at[didx.at[k0]], add=True)

            @pl.when(k0 + 2 < nch)
            def _():
                pltpu.async_copy(hp_hbm.at[sidx.at[k0 + 2]], rows0, sem0)

            pltpu.make_async_copy(hp_hbm.at[sidx.at[k0 + 1]], rows1, sem1).wait()
            pltpu.sync_copy(rows1, acc.at[didx.at[k0 + 1]], add=True)
            return carry

        lax.fori_loop(0, nch // 2, pair, 0)
        if nch % 2:
            pltpu.make_async_copy(hp_hbm.at[sidx.at[nch - 1]], rows0, sem0).wait()
            pltpu.sync_copy(rows0, acc.at[didx.at[nch - 1]], add=True)

    def blk(j, carry):
        process(_BLK_CH)

        @pl.when(j + 1 < _NFULL)
        def _():
            j0 = pl.multiple_of((j + 1) * _BLK_CH, _BLK_CH)
            pltpu.sync_copy(src_hbm.at[w, pl.ds(j0, _BLK_CH)], sidx)
            pltpu.sync_copy(dst_hbm.at[w, pl.ds(j0, _BLK_CH)], didx)
        return carry

    lax.fori_loop(0, _NFULL, blk, 0)
    pltpu.sync_copy(src_hbm.at[w, pl.ds(_NFULL * _BLK_CH, _TAIL)],
                    sidx.at[pl.ds(0, _TAIL)])
    pltpu.sync_copy(dst_hbm.at[w, pl.ds(_NFULL * _BLK_CH, _TAIL)],
                    didx.at[pl.ds(0, _TAIL)])
    process(_TAIL)

    plsc.subcore_barrier()
    pltpu.sync_copy(acc.at[pl.ds(row_base, _ROWS_PER_TILE)],
                    out_hbm.at[c, pl.ds(row_base, _ROWS_PER_TILE)])


@functools.lru_cache(maxsize=None)
def _get_sc_scatter():
    mesh = plsc.VectorSubcoreMesh(core_axis_name="c", subcore_axis_name="s")
    return pl.kernel(
        _sc_scatter_body,
        out_type=jax.ShapeDtypeStruct((_NC, _N_PAD, _D), jnp.float32),
        mesh=mesh,
        scratch_types=[
            pltpu.VMEM((_BLK_CH, _CHUNK), jnp.int32),
            pltpu.VMEM((_BLK_CH, _CHUNK), jnp.int32),
            pltpu.VMEM((_CHUNK, _D), jnp.float32),
            pltpu.VMEM((_CHUNK, _D), jnp.float32),
            pltpu.VMEM_SHARED((_N_PAD, _D), jnp.float32),
            pltpu.SemaphoreType.DMA,
            pltpu.SemaphoreType.DMA,
            pltpu.SemaphoreType.DMA,
        ],
    )


_BLK = 512
_GRID = _N_PAD // _BLK


def _tc1_body(degp_ref, x_ref, w1_ref, hp_ref, dinv_ref):
    deg = degp_ref[0, :, 0:1] + degp_ref[1, :, 0:1] + 1.0
    dinv = lax.rsqrt(deg)
    h = jnp.dot(x_ref[...], w1_ref[...], preferred_element_type=jnp.float32)
    hp_ref[...] = dinv * h
    dinv_ref[...] = dinv


def _tc2_body(parts_ref, hp_ref, dinv_ref, b1_ref, w2_ref, hp2_ref):
    dinv = dinv_ref[...]
    agg = parts_ref[0] + parts_ref[1] + hp_ref[...]
    o1 = jnp.maximum(dinv * agg + b1_ref[...], 0.0)
    hp2_ref[...] = dinv * jnp.dot(o1, w2_ref[...], preferred_element_type=jnp.float32)


def _tc3_body(parts_ref, hp_ref, dinv_ref, b2_ref, wf1_ref, bf1_ref,
              wf2_ref, bf2_ref, y_ref):
    dinv = dinv_ref[...]
    agg = parts_ref[0] + parts_ref[1] + hp_ref[...]
    o2 = jnp.maximum(dinv * agg + b2_ref[...], 0.0)
    h3 = jnp.maximum(
        jnp.dot(o2, wf1_ref[...], preferred_element_type=jnp.float32) + bf1_ref[...],
        0.0)
    y_ref[...] = jnp.dot(h3, wf2_ref[...], preferred_element_type=jnp.float32) + bf2_ref[...]


def _row_blk(*trail):
    return pl.BlockSpec((_BLK,) + trail, lambda i: (i,) + (0,) * len(trail))


def _parts_blk(width):
    return pl.BlockSpec((_NC, _BLK, width), lambda i: (0, i, 0))


def _full(shape):
    return pl.BlockSpec(shape, lambda i: (0,) * len(shape))


_tc1 = pl.pallas_call(
    _tc1_body,
    grid=(_GRID,),
    in_specs=[_parts_blk(_D), _row_blk(_D), _full((_D, _D))],
    out_specs=[_row_blk(_D), _row_blk(1)],
    out_shape=[jax.ShapeDtypeStruct((_N_PAD, _D), jnp.float32),
               jax.ShapeDtypeStruct((_N_PAD, 1), jnp.float32)],
)

_tc2 = pl.pallas_call(
    _tc2_body,
    grid=(_GRID,),
    in_specs=[_parts_blk(_D), _row_blk(_D), _row_blk(1), _full((1, _D)),
              _full((_D, _D))],
    out_specs=_row_blk(_D),
    out_shape=jax.ShapeDtypeStruct((_N_PAD, _D), jnp.float32),
)

_tc3 = pl.pallas_call(
    _tc3_body,
    grid=(_GRID,),
    in_specs=[_parts_blk(_D), _row_blk(_D), _row_blk(1), _full((1, _D)),
              _full((_D, 64)), _full((1, 64)), _full((64, 1)), _full((1, 1))],
    out_specs=_row_blk(1),
    out_shape=jax.ShapeDtypeStruct((_N_PAD, 1), jnp.float32),
)


def kernel(x, edge_index, W1, b1, W2, b2, Wf1, bf1, Wf2, bf2):
    src4 = edge_index[0].reshape(_NC * _NS, _STEPS, _CHUNK)
    dst4 = edge_index[1].reshape(_NC * _NS, _STEPS, _CHUNK)
    zeros_big = jnp.zeros((_N_PAD, _D), jnp.float32)
    ones_tab = jnp.ones((_N_PAD, _D), jnp.float32)

    x_pad = jnp.zeros((_N_PAD, _D), jnp.float32).at[:_N].set(x)
    deg_parts = _get_sc_scatter()(ones_tab, src4, dst4, zeros_big)
    hp1, dinv = _tc1(deg_parts, x_pad, W1)
    a1 = _get_sc_scatter()(hp1, src4, dst4, zeros_big)
    hp2 = _tc2(a1, hp1, dinv, b1.reshape(1, _D), W2)
    a2 = _get_sc_scatter()(hp2, src4, dst4, zeros_big)
    y = _tc3(a2, hp2, dinv, b2.reshape(1, _D), Wf1, bf1.reshape(1, 64),
             Wf2, bf2.reshape(1, 1))
    return y[:_N]
